# baseline (speedup 1.0000x reference)
import functools

import jax
import jax.numpy as jnp
from jax import lax
from jax.experimental import pallas as pl
from jax.experimental.pallas import tpu as pltpu
from jax.experimental.pallas import tpu_sc as plsc

DIM = 1024
INTER = 512
N_EXPERTS = 8
N_SHARED = 2
T = 2048
TB = 256
S_INTER = INTER * N_SHARED
B = 256
NB = (2 * T) // B + N_EXPERTS
NPAD = NB * B
NC = 2
NS = 16
NW = NC * NS


def _sc_mesh():
    return plsc.VectorSubcoreMesh(
        core_axis_name="c", subcore_axis_name="s", num_cores=NC,
        num_subcores=NS)


def _wid():
    return lax.axis_index("s") * NC + lax.axis_index("c")


def _gate_shared_body(x_ref, gw_ref, sw1_ref, sw2_ref, sw3_ref,
                      ei_ref, wv_ref, ys_ref):
    x = x_ref[...]
    logits = lax.dot_general(x, gw_ref[...], (((1,), (1,)), ((), ())),
                             preferred_element_type=jnp.float32)
    s = jax.nn.sigmoid(logits)
    iota = lax.broadcasted_iota(jnp.int32, s.shape, 1)
    m1 = jnp.max(s, axis=1, keepdims=True)
    i1 = jnp.min(jnp.where(s == m1, iota, N_EXPERTS), axis=1, keepdims=True)
    s2 = jnp.where(iota == i1, -jnp.inf, s)
    m2 = jnp.max(s2, axis=1, keepdims=True)
    i2 = jnp.min(jnp.where(s2 == m2, iota, N_EXPERTS), axis=1, keepdims=True)
    denom = m1 + m2
    wa = m1 / denom
    wb = m2 / denom
    ei_ref[...] = (jnp.where(iota == 0, i1, 0)
                   + jnp.where(iota == 1, i2, 0)).astype(jnp.int32)
    wv_ref[...] = (jnp.where(iota == 0, wa, 0.0)
                   + jnp.where(iota == 1, wb, 0.0))
    h1 = lax.dot_general(x, sw1_ref[...], (((1,), (1,)), ((), ())),
                         preferred_element_type=jnp.float32)
    h3 = lax.dot_general(x, sw3_ref[...], (((1,), (1,)), ((), ())),
                         preferred_element_type=jnp.float32)
    h = h1 * jax.nn.sigmoid(h1) * h3
    ys_ref[...] = lax.dot_general(h, sw2_ref[...], (((1,), (1,)), ((), ())),
                                  preferred_element_type=jnp.float32)


def _gate_shared(xf, gate_w, sw1, sw2, sw3):
    nb = T // TB
    return pl.pallas_call(
        _gate_shared_body,
        grid=(nb,),
        in_specs=[
            pl.BlockSpec((TB, DIM), lambda i: (i, 0)),
            pl.BlockSpec((N_EXPERTS, DIM), lambda i: (0, 0)),
            pl.BlockSpec((S_INTER, DIM), lambda i: (0, 0)),
            pl.BlockSpec((DIM, S_INTER), lambda i: (0, 0)),
            pl.BlockSpec((S_INTER, DIM), lambda i: (0, 0)),
        ],
        out_specs=[
            pl.BlockSpec((TB, N_EXPERTS), lambda i: (i, 0)),
            pl.BlockSpec((TB, N_EXPERTS), lambda i: (i, 0)),
            pl.BlockSpec((TB, DIM), lambda i: (i, 0)),
        ],
        out_shape=[
            jax.ShapeDtypeStruct((T, N_EXPERTS), jnp.int32),
            jax.ShapeDtypeStruct((T, N_EXPERTS), jnp.float32),
            jax.ShapeDtypeStruct((T, DIM), jnp.float32),
        ],
    )(xf, gate_w, sw1, sw2, sw3)


def _meta_body(ei_ref, posb_ref, meta_ref):
    e1 = ei_ref[:, 0:1]
    e2 = ei_ref[:, 1:2]
    iota8 = lax.broadcasted_iota(jnp.int32, (T, N_EXPERTS), 1)
    sel1 = iota8 == e1
    sel2 = iota8 == e2
    selm = jnp.where(sel1 | sel2, 1.0, 0.0)
    r = lax.broadcasted_iota(jnp.int32, (T, T), 0)
    c = lax.broadcasted_iota(jnp.int32, (T, T), 1)
    tri = jnp.where(r >= c, 1.0, 0.0)
    csum = lax.dot_general(tri, selm, (((1,), (0,)), ((), ())),
                           preferred_element_type=jnp.float32)
    cnt = csum[T - 1:T, :]
    nblk = jnp.floor((cnt + (B - 1)) * (1.0 / B))
    r8 = lax.broadcasted_iota(jnp.int32, (N_EXPERTS, N_EXPERTS), 0)
    c8 = lax.broadcasted_iota(jnp.int32, (N_EXPERTS, N_EXPERTS), 1)
    strict = jnp.where(r8 < c8, 1.0, 0.0)
    blkbase = lax.dot_general(nblk, strict, (((1,), (0,)), ((), ())),
                              preferred_element_type=jnp.float32)
    pos = blkbase * float(B) + csum - 1.0
    pos1 = jnp.sum(jnp.where(sel1, pos, 0.0), axis=1, keepdims=True)
    pos2 = jnp.sum(jnp.where(sel2, pos, 0.0), axis=1, keepdims=True)
    posb_ref[...] = (jnp.where(iota8 == 0, pos1, 0.0)
                     + jnp.where(iota8 == 1, pos2, 0.0)).astype(jnp.int32)
    ii = lax.broadcasted_iota(jnp.int32, (128, N_EXPERTS), 0).astype(jnp.float32)
    eidcol = jnp.sum(jnp.where(blkbase <= ii, 1.0, 0.0), axis=1,
                     keepdims=True) - 1.0
    nused = jnp.sum(nblk, axis=1, keepdims=True)
    i8 = lax.broadcasted_iota(jnp.int32, (128, N_EXPERTS), 1)
    meta_ref[...] = (jnp.where(i8 == 0, eidcol, 0.0)
                     + jnp.where(i8 == 1, nused, 0.0)).astype(jnp.int32)


def _meta(ei):
    return pl.pallas_call(
        _meta_body,
        grid=(1,),
        in_specs=[pl.BlockSpec((T, N_EXPERTS), lambda i: (0, 0))],
        out_specs=[
            pl.BlockSpec((T, N_EXPERTS), lambda i: (0, 0)),
            pl.BlockSpec((128, N_EXPERTS), lambda i: (0, 0)),
        ],
        out_shape=[
            jax.ShapeDtypeStruct((T, N_EXPERTS), jnp.int32),
            jax.ShapeDtypeStruct((128, N_EXPERTS), jnp.int32),
        ],
    )(ei)


def _sc_scatter_routing(pos1, pos2, wa, wb):
    def body(p1_hbm, p2_hbm, wa_hbm, wb_hbm, tok_hbm, wrow_hbm,
             tok_v, wrow_v, pos_v, w_v):
        @pl.when(_wid() == 0)
        def _():
            def init(i, carry):
                tok_v[pl.ds(i * 16, 16)] = jnp.zeros((16,), jnp.int32)
                wrow_v[pl.ds(i * 16, 16)] = jnp.zeros((16,), jnp.float32)
                return carry
            lax.fori_loop(0, NPAD // 16, init, 0)
            for p_hbm, wx_hbm in ((p1_hbm, wa_hbm), (p2_hbm, wb_hbm)):
                pltpu.sync_copy(p_hbm, pos_v)
                pltpu.sync_copy(wx_hbm, w_v)

                def step(i, carry):
                    idx = pos_v[pl.ds(i * 16, 16)]
                    tvals = lax.iota(jnp.int32, 16) + i * 16
                    plsc.store_scatter(tok_v, [idx], tvals)
                    wv = w_v[pl.ds(i * 16, 16)]
                    plsc.store_scatter(wrow_v, [idx], wv)
                    return carry
                lax.fori_loop(0, T // 16, step, 0)
            pltpu.sync_copy(tok_v, tok_hbm)
            pltpu.sync_copy(wrow_v, wrow_hbm)

    fn = pl.kernel(
        body,
        out_type=[
            jax.ShapeDtypeStruct((NPAD,), jnp.int32),
            jax.ShapeDtypeStruct((NPAD,), jnp.float32),
        ],
        mesh=_sc_mesh(),
        scratch_types=[
            pltpu.VMEM((NPAD,), jnp.int32),
            pltpu.VMEM((NPAD,), jnp.float32),
            pltpu.VMEM((T,), jnp.int32),
            pltpu.VMEM((T,), jnp.float32),
        ],
        compiler_params=pltpu.CompilerParams(needs_layout_passes=False),
    )
    return fn(pos1, pos2, wa, wb)


def _sc_gather(tok, xf):
    rows_per_tile = NPAD // NW
    chunk = 48
    nchunks = rows_per_tile // chunk

    def body(tok_hbm, xf_hbm, xg_hbm, idx_v, rows_v, sem):
        w = _wid()

        def step(ci, carry):
            base = w * rows_per_tile + ci * chunk
            pltpu.sync_copy(tok_hbm.at[pl.ds(base, chunk)], idx_v)
            pltpu.async_copy(xf_hbm.at[idx_v], rows_v, sem).wait()
            pltpu.sync_copy(rows_v, xg_hbm.at[pl.ds(base, chunk)])
            return carry
        lax.fori_loop(0, nchunks, step, 0)

    fn = pl.kernel(
        body,
        out_type=jax.ShapeDtypeStruct((NPAD, DIM), jnp.float32),
        mesh=_sc_mesh(),
        scratch_types=[
            pltpu.VMEM((chunk,), jnp.int32),
            pltpu.VMEM((chunk, DIM), jnp.float32),
            pltpu.SemaphoreType.DMA,
        ],
    )
    return fn(tok, xf)


def _gemm_body(eid_ref, nu_ref, xg_ref, wrow_ref, ew1_ref, ew2_ref, ew3_ref,
               o_ref):
    i = pl.program_id(0)

    @pl.when(i < nu_ref[0])
    def _():
        x = xg_ref[...]
        h1 = lax.dot_general(x, ew1_ref[0], (((1,), (1,)), ((), ())),
                             preferred_element_type=jnp.float32)
        h3 = lax.dot_general(x, ew3_ref[0], (((1,), (1,)), ((), ())),
                             preferred_element_type=jnp.float32)
        h = h1 * jax.nn.sigmoid(h1) * h3
        y = lax.dot_general(h, ew2_ref[0], (((1,), (1,)), ((), ())),
                            preferred_element_type=jnp.float32)
        o_ref[...] = y * wrow_ref[0]

    @pl.when(i >= nu_ref[0])
    def _():
        o_ref[...] = jnp.zeros_like(o_ref)


def _gemm(eid, nused, xg, wrow3, ew1, ew2, ew3):
    return pl.pallas_call(
        _gemm_body,
        grid_spec=pltpu.PrefetchScalarGridSpec(
            num_scalar_prefetch=2,
            grid=(NB,),
            in_specs=[
                pl.BlockSpec((B, DIM), lambda i, eid, nu: (i, 0)),
                pl.BlockSpec((1, B, 1), lambda i, eid, nu: (i, 0, 0)),
                pl.BlockSpec((1, INTER, DIM), lambda i, eid, nu: (eid[i], 0, 0)),
                pl.BlockSpec((1, DIM, INTER), lambda i, eid, nu: (eid[i], 0, 0)),
                pl.BlockSpec((1, INTER, DIM), lambda i, eid, nu: (eid[i], 0, 0)),
            ],
            out_specs=pl.BlockSpec((B, DIM), lambda i, eid, nu: (i, 0)),
        ),
        out_shape=jax.ShapeDtypeStruct((NPAD, DIM), jnp.float32),
    )(eid, nused, xg, wrow3, ew1, ew2, ew3)


def _sc_combine(pos1, pos2, yg, ys):
    tok_per_tile = T // NW
    chunk = 16
    nchunks = tok_per_tile // chunk

    def body(p1_hbm, p2_hbm, yg_hbm, ys_hbm, y_hbm,
             i1_v, i2_v, b1_v, b2_v, bs_v, sem1, sem2):
        w = _wid()

        def step(ci, carry):
            tbase = w * tok_per_tile + ci * chunk
            pltpu.sync_copy(p1_hbm.at[pl.ds(tbase, chunk)], i1_v)
            pltpu.sync_copy(p2_hbm.at[pl.ds(tbase, chunk)], i2_v)
            cp1 = pltpu.async_copy(yg_hbm.at[i1_v], b1_v, sem1)
            cp2 = pltpu.async_copy(yg_hbm.at[i2_v], b2_v, sem2)
            pltpu.sync_copy(ys_hbm.at[pl.ds(tbase, chunk)], bs_v)
            cp1.wait()
            cp2.wait()

            def add_row(rr, carry2):
                def add_col(jc, carry3):
                    sl = pl.ds(jc * 16, 16)
                    bs_v[rr, sl] = bs_v[rr, sl] + b1_v[rr, sl] + b2_v[rr, sl]
                    return carry3
                lax.fori_loop(0, DIM // 16, add_col, 0, unroll=4)
                return carry2
            lax.fori_loop(0, chunk, add_row, 0)
            pltpu.sync_copy(bs_v, y_hbm.at[pl.ds(tbase, chunk)])
            return carry
        lax.fori_loop(0, nchunks, step, 0)

    fn = pl.kernel(
        body,
        out_type=jax.ShapeDtypeStruct((T, DIM), jnp.float32),
        mesh=_sc_mesh(),
        scratch_types=[
            pltpu.VMEM((chunk,), jnp.int32),
            pltpu.VMEM((chunk,), jnp.int32),
            pltpu.VMEM((chunk, DIM), jnp.float32),
            pltpu.VMEM((chunk, DIM), jnp.float32),
            pltpu.VMEM((chunk, DIM), jnp.float32),
            pltpu.SemaphoreType.DMA,
            pltpu.SemaphoreType.DMA,
        ],
    )
    return fn(pos1, pos2, yg, ys)


def kernel(x, gate_w, ew1, ew2, ew3, sw1, sw2, sw3):
    shape = x.shape
    xf = x.reshape(-1, DIM)

    ei, wv, ys = _gate_shared(xf, gate_w, sw1, sw2, sw3)
    posb, meta = _meta(ei)

    pos1 = posb[:, 0]
    pos2 = posb[:, 1]
    wa = wv[:, 0]
    wb = wv[:, 1]
    eid = meta[:NB, 0]
    nused = meta[0:1, 1]

    tok, wrow = _sc_scatter_routing(pos1, pos2, wa, wb)
    xg = _sc_gather(tok, xf)
    yg = _gemm(eid, nused, xg, wrow.reshape(NB, B, 1), ew1, ew2, ew3)
    y = _sc_combine(pos1, pos2, yg, ys)
    return y.reshape(shape)
